# baseline (speedup 1.0000x reference)
import functools
import math

import jax
import jax.numpy as jnp
from jax import lax
from jax.experimental import pallas as pl
from jax.experimental.pallas import tpu as pltpu
from jax.experimental.pallas import tpu_sc as plsc

N_CLUSTERS = 32
DELTA_DIST = 1.5
W_VAR, W_DIST, W_REG, W_SMOOTH, W_SEED = 3.0, 1.0, 0.001, 5.0, 5.0

_LOG_LO = math.log(1e-12)
_PMAX = 1.0 - 1e-12

_NC, _NS = 2, 16
_NW = _NC * _NS
_NIDX = 128


def _sum11(a):
    return jnp.sum(jnp.sum(a, axis=0, keepdims=True), axis=1, keepdims=True)


def _sc_feature_sums(features, lab3):
    n = features.shape[0]
    chunk = n // _NW
    nstream = chunk // _NIDX
    C = N_CLUSTERS
    mesh = plsc.VectorSubcoreMesh(
        core_axis_name="c", subcore_axis_name="s", num_cores=_NC)

    @functools.partial(
        pl.kernel, mesh=mesh,
        compiler_params=pltpu.CompilerParams(use_tc_tiling_on_sc=False),
        out_type=jax.ShapeDtypeStruct((_NC * C, 16), jnp.float32),
        scratch_types=[
            pltpu.VMEM((chunk, 16), jnp.float32),
            pltpu.VMEM((nstream, _NIDX), jnp.int32),
            pltpu.VMEM((C, 16), jnp.float32),
            pltpu.VMEM_SHARED((C, 16), jnp.float32),
        ],
    )
    def k(x_hbm, lab_hbm, xpart_hbm, x_v, lab_v, zbuf, accx_sh):
        cidx = lax.axis_index("c")
        sidx = lax.axis_index("s")
        wid = cidx * _NS + sidx
        base = wid * chunk

        pltpu.sync_copy(x_hbm.at[pl.ds(base, chunk)], x_v)
        pltpu.sync_copy(lab_hbm.at[wid], lab_v)

        zero16 = jnp.zeros((16,), jnp.float32)
        for r in range(C):
            zbuf[r, :] = zero16

        @pl.when(sidx == 0)
        def _():
            pltpu.sync_copy(zbuf, accx_sh)

        plsc.subcore_barrier()
        for kk in range(nstream):
            pltpu.sync_copy(x_v.at[pl.ds(kk * _NIDX, _NIDX)],
                            accx_sh.at[lab_v.at[kk]], add=True)

        plsc.subcore_barrier()

        @pl.when(sidx == 0)
        def _():
            pltpu.sync_copy(accx_sh, xpart_hbm.at[pl.ds(cidx * C, C)])

    return k(features, lab3)


def _loss_kernel(x_ref, st_ref, lab_ref, xpart_ref,
                 loss_ref, varl_ref, distl_ref, regl_ref, seedl_ref, smoothl_ref,
                 acc_e, cm_s, c2_s, i2v_s, row_bce, row_seed,
                 *, nblocks, n_total):
    phase = pl.program_id(0)
    j = pl.program_id(1)
    C = N_CLUSTERS
    f32 = jnp.float32

    labels = lab_ref[...]
    B = labels.shape[1]
    cid = lax.broadcasted_iota(jnp.int32, (C, B), 0)
    onehot = (cid == labels)

    @pl.when(jnp.logical_and(phase == 0, j == 0))
    def _init():
        acc_e[...] = jnp.zeros_like(acc_e)

    @pl.when(phase == 0)
    def _estats():
        e = jnp.exp(st_ref[1:2, :])
        ones_b = jnp.ones((1, B), dtype=f32)
        e3 = jnp.concatenate([ones_b, e, e * e], axis=0)
        oh = onehot.astype(f32)
        acc_e[...] += lax.dot_general(
            oh, e3, (((1,), (1,)), ((), ())), preferred_element_type=f32)

    @pl.when(jnp.logical_and(phase == 1, j == 0))
    def _finalize_stats():
        cnt = acc_e[:, 0:1]
        sx = xpart_ref[0:C, :] + xpart_ref[C:2 * C, :]
        cm = sx / cnt
        cm_s[...] = cm
        c2_s[...] = jnp.sum(cm * cm, axis=1, keepdims=True)
        var_c = acc_e[:, 1:2] / cnt
        i2v_s[...] = 0.5 / var_c

    @pl.when(phase == 1)
    def _dense():
        x = x_ref[...]
        cm = cm_s[...]
        xc = lax.dot_general(
            cm, x, (((1,), (0,)), ((), ())), preferred_element_type=f32)
        xsq = x * x
        ones_d = jnp.ones((1, x.shape[0]), dtype=f32)
        x2 = lax.dot_general(
            ones_d, xsq, (((1,), (0,)), ((), ())), preferred_element_type=f32)
        d2 = jnp.maximum(x2 - 2.0 * xc + c2_s[...], 0.0)
        t = d2 * i2v_s[...]
        prob = jnp.exp(-t)
        p = jnp.minimum(prob, _PMAX)
        log1mp = jnp.log1p(-p)
        fold8 = (log1mp[0:8, :] + log1mp[8:16, :]
                 + log1mp[16:24, :] + log1mp[24:32, :])
        prob_own = jnp.sum(jnp.where(onehot, prob, 0.0), axis=0, keepdims=True)
        lp_own = jnp.maximum(jnp.log(prob_own), _LOG_LO)
        l1mp_own = jnp.log1p(-jnp.minimum(prob_own, _PMAX))
        keep = j != 0
        row_bce[...] = jnp.where(keep, row_bce[...], 0.0) + fold8
        row_bce[0:1, :] += lp_own - l1mp_own
        dsq = prob_own - st_ref[0:1, :]
        row_seed[...] = jnp.where(keep, row_seed[...], 0.0) + dsq * dsq

    @pl.when(jnp.logical_and(phase == 1, j == nblocks - 1))
    def _combine():
        n = f32(n_total)
        d = x_ref.shape[0]
        cnt = acc_e[:, 0:1]
        cm = cm_s[...]
        c2col = c2_s[...]
        cmsq = cm * cm
        onesd = jnp.ones((1, d), dtype=f32)
        c2row = lax.dot_general(
            onesd, cmsq, (((1,), (1,)), ((), ())), preferred_element_type=f32)
        gram = lax.dot_general(
            cm, cm, (((1,), (1,)), ((), ())), preferred_element_type=f32)
        rr = lax.broadcasted_iota(jnp.int32, (C, C), 0)
        cc = lax.broadcasted_iota(jnp.int32, (C, C), 1)
        eye = (rr == cc).astype(f32)
        dmat = jnp.sqrt(jnp.maximum(c2col + c2row - 2.0 * gram, 0.0) + eye)
        hinge = jnp.maximum(2.0 * DELTA_DIST - dmat, 0.0)
        dist_loss = _sum11(hinge * hinge * (1.0 - eye)) / f32((C - 1) * C)
        reg_loss = _sum11(jnp.sqrt(c2col)) / f32(C)
        var_c = acc_e[:, 1:2] / cnt
        smooth_c = acc_e[:, 2:3] / cnt - var_c * var_c
        smoothness_loss = _sum11(smooth_c) / f32(C)
        var_loss = -_sum11(row_bce[...]) / (n * f32(C))
        seed_loss = _sum11(row_seed[...]) / n
        loss = (W_VAR * var_loss + W_DIST * dist_loss + W_REG * reg_loss
                + W_SMOOTH * smoothness_loss + W_SEED * seed_loss)
        loss_ref[...] = loss
        varl_ref[...] = W_VAR * var_loss
        distl_ref[...] = W_DIST * dist_loss
        regl_ref[...] = W_REG * reg_loss
        seedl_ref[...] = W_SEED * seed_loss
        smoothl_ref[...] = W_SMOOTH * smoothness_loss


@jax.jit
def kernel(features, seediness, group_labels):
    n, d = features.shape
    blk = 16384
    nblocks = n // blk
    C = N_CLUSTERS
    chunk = n // _NW
    nstream = chunk // _NIDX

    lab_i32 = group_labels.astype(jnp.int32)
    lab3 = lab_i32.reshape(_NW, nstream, _NIDX)
    xpart = _sc_feature_sums(features, lab3)

    xt = features.T
    st = seediness.T
    lab = lab_i32.reshape(1, n)

    scalar = jax.ShapeDtypeStruct((1, 1), jnp.float32)
    out = pl.pallas_call(
        functools.partial(_loss_kernel, nblocks=nblocks, n_total=n),
        grid=(2, nblocks),
        in_specs=[
            pl.BlockSpec((d, blk), lambda p, j: (0, j * p)),
            pl.BlockSpec((2, blk), lambda p, j: (0, j)),
            pl.BlockSpec((1, blk), lambda p, j: (0, j)),
            pl.BlockSpec((2 * C, 16), lambda p, j: (0, 0)),
        ],
        out_specs=[pl.BlockSpec((1, 1), lambda p, j: (0, 0))] * 6,
        out_shape=[scalar] * 6,
        scratch_shapes=[
            pltpu.VMEM((C, 3), jnp.float32),
            pltpu.VMEM((C, 16), jnp.float32),
            pltpu.VMEM((C, 1), jnp.float32),
            pltpu.VMEM((C, 1), jnp.float32),
            pltpu.VMEM((8, blk), jnp.float32),
            pltpu.VMEM((1, blk), jnp.float32),
        ],
    )(xt, st, lab, xpart)
    return tuple(o.reshape(()) for o in out)


# speedup vs baseline: 1.0110x; 1.0110x over previous
import functools
import math

import jax
import jax.numpy as jnp
from jax import lax
from jax.experimental import pallas as pl
from jax.experimental.pallas import tpu as pltpu
from jax.experimental.pallas import tpu_sc as plsc

N_CLUSTERS = 32
DELTA_DIST = 1.5
W_VAR, W_DIST, W_REG, W_SMOOTH, W_SEED = 3.0, 1.0, 0.001, 5.0, 5.0

_LOG_LO = math.log(1e-12)
_PMAX = 1.0 - 1e-12

_NC, _NS = 2, 16
_NW = _NC * _NS
_NIDX = 128


def _sum11(a):
    return jnp.sum(jnp.sum(a, axis=0, keepdims=True), axis=1, keepdims=True)


def _sc_feature_sums(features, lab3):
    n = features.shape[0]
    chunk = n // _NW
    nstream = chunk // _NIDX
    C = N_CLUSTERS
    mesh = plsc.VectorSubcoreMesh(
        core_axis_name="c", subcore_axis_name="s", num_cores=_NC)

    @functools.partial(
        pl.kernel, mesh=mesh,
        compiler_params=pltpu.CompilerParams(use_tc_tiling_on_sc=False),
        out_type=jax.ShapeDtypeStruct((_NC * C, 16), jnp.float32),
        scratch_types=[
            pltpu.VMEM((chunk, 16), jnp.float32),
            pltpu.VMEM((nstream, _NIDX), jnp.int32),
            pltpu.VMEM((C, 16), jnp.float32),
            pltpu.VMEM_SHARED((C, 16), jnp.float32),
            pltpu.SemaphoreType.DMA,
        ],
    )
    def k(x_hbm, lab_hbm, xpart_hbm, x_v, lab_v, zbuf, accx_sh, sem):
        cidx = lax.axis_index("c")
        sidx = lax.axis_index("s")
        wid = cidx * _NS + sidx
        base = wid * chunk

        pltpu.sync_copy(x_hbm.at[pl.ds(base, chunk)], x_v)
        pltpu.sync_copy(lab_hbm.at[wid], lab_v)

        zero16 = jnp.zeros((16,), jnp.float32)
        for r in range(C):
            zbuf[r, :] = zero16

        @pl.when(sidx == 0)
        def _():
            pltpu.sync_copy(zbuf, accx_sh)

        plsc.subcore_barrier()
        copies = [
            pltpu.async_copy(x_v.at[pl.ds(kk * _NIDX, _NIDX)],
                             accx_sh.at[lab_v.at[kk]], sem, add=True)
            for kk in range(nstream)
        ]
        for cp in copies:
            cp.wait()

        plsc.subcore_barrier()

        @pl.when(sidx == 0)
        def _():
            pltpu.sync_copy(accx_sh, xpart_hbm.at[pl.ds(cidx * C, C)])

    return k(features, lab3)


def _loss_kernel(x_ref, st_ref, lab_ref, xpart_ref,
                 loss_ref, varl_ref, distl_ref, regl_ref, seedl_ref, smoothl_ref,
                 acc_e, cm_s, c2_s, i2v_s, row_bce, row_seed,
                 *, nblocks, n_total):
    phase = pl.program_id(0)
    j = pl.program_id(1)
    C = N_CLUSTERS
    f32 = jnp.float32

    labels = lab_ref[...]
    B = labels.shape[1]
    cid = lax.broadcasted_iota(jnp.int32, (C, B), 0)
    onehot = (cid == labels)

    @pl.when(jnp.logical_and(phase == 0, j == 0))
    def _init():
        acc_e[...] = jnp.zeros_like(acc_e)

    @pl.when(phase == 0)
    def _estats():
        e = jnp.exp(st_ref[1:2, :])
        ones_b = jnp.ones((1, B), dtype=f32)
        e3 = jnp.concatenate([ones_b, e, e * e], axis=0)
        oh = onehot.astype(f32)
        acc_e[...] += lax.dot_general(
            oh, e3, (((1,), (1,)), ((), ())), preferred_element_type=f32)

    @pl.when(jnp.logical_and(phase == 1, j == 0))
    def _finalize_stats():
        cnt = acc_e[:, 0:1]
        sx = xpart_ref[0:C, :] + xpart_ref[C:2 * C, :]
        cm = sx / cnt
        cm_s[...] = cm
        c2_s[...] = jnp.sum(cm * cm, axis=1, keepdims=True)
        var_c = acc_e[:, 1:2] / cnt
        i2v_s[...] = 0.5 / var_c

    @pl.when(phase == 1)
    def _dense():
        x = x_ref[...]
        cm = cm_s[...]
        xc = lax.dot_general(
            cm, x, (((1,), (0,)), ((), ())), preferred_element_type=f32)
        xsq = x * x
        ones_d = jnp.ones((1, x.shape[0]), dtype=f32)
        x2 = lax.dot_general(
            ones_d, xsq, (((1,), (0,)), ((), ())), preferred_element_type=f32)
        d2 = jnp.maximum(x2 - 2.0 * xc + c2_s[...], 0.0)
        t = d2 * i2v_s[...]
        prob = jnp.exp(-t)
        p = jnp.minimum(prob, _PMAX)
        log1mp = jnp.log1p(-p)
        fold8 = (log1mp[0:8, :] + log1mp[8:16, :]
                 + log1mp[16:24, :] + log1mp[24:32, :])
        prob_own = jnp.sum(jnp.where(onehot, prob, 0.0), axis=0, keepdims=True)
        lp_own = jnp.maximum(jnp.log(prob_own), _LOG_LO)
        l1mp_own = jnp.log1p(-jnp.minimum(prob_own, _PMAX))
        keep = j != 0
        row_bce[...] = jnp.where(keep, row_bce[...], 0.0) + fold8
        row_bce[0:1, :] += lp_own - l1mp_own
        dsq = prob_own - st_ref[0:1, :]
        row_seed[...] = jnp.where(keep, row_seed[...], 0.0) + dsq * dsq

    @pl.when(jnp.logical_and(phase == 1, j == nblocks - 1))
    def _combine():
        n = f32(n_total)
        d = x_ref.shape[0]
        cnt = acc_e[:, 0:1]
        cm = cm_s[...]
        c2col = c2_s[...]
        cmsq = cm * cm
        onesd = jnp.ones((1, d), dtype=f32)
        c2row = lax.dot_general(
            onesd, cmsq, (((1,), (1,)), ((), ())), preferred_element_type=f32)
        gram = lax.dot_general(
            cm, cm, (((1,), (1,)), ((), ())), preferred_element_type=f32)
        rr = lax.broadcasted_iota(jnp.int32, (C, C), 0)
        cc = lax.broadcasted_iota(jnp.int32, (C, C), 1)
        eye = (rr == cc).astype(f32)
        dmat = jnp.sqrt(jnp.maximum(c2col + c2row - 2.0 * gram, 0.0) + eye)
        hinge = jnp.maximum(2.0 * DELTA_DIST - dmat, 0.0)
        dist_loss = _sum11(hinge * hinge * (1.0 - eye)) / f32((C - 1) * C)
        reg_loss = _sum11(jnp.sqrt(c2col)) / f32(C)
        var_c = acc_e[:, 1:2] / cnt
        smooth_c = acc_e[:, 2:3] / cnt - var_c * var_c
        smoothness_loss = _sum11(smooth_c) / f32(C)
        var_loss = -_sum11(row_bce[...]) / (n * f32(C))
        seed_loss = _sum11(row_seed[...]) / n
        loss = (W_VAR * var_loss + W_DIST * dist_loss + W_REG * reg_loss
                + W_SMOOTH * smoothness_loss + W_SEED * seed_loss)
        loss_ref[...] = loss
        varl_ref[...] = W_VAR * var_loss
        distl_ref[...] = W_DIST * dist_loss
        regl_ref[...] = W_REG * reg_loss
        seedl_ref[...] = W_SEED * seed_loss
        smoothl_ref[...] = W_SMOOTH * smoothness_loss


@jax.jit
def kernel(features, seediness, group_labels):
    n, d = features.shape
    blk = 16384
    nblocks = n // blk
    C = N_CLUSTERS
    chunk = n // _NW
    nstream = chunk // _NIDX

    lab_i32 = group_labels.astype(jnp.int32)
    lab3 = lab_i32.reshape(_NW, nstream, _NIDX)
    xpart = _sc_feature_sums(features, lab3)

    xt = features.T
    st = seediness.T
    lab = lab_i32.reshape(1, n)

    scalar = jax.ShapeDtypeStruct((1, 1), jnp.float32)
    out = pl.pallas_call(
        functools.partial(_loss_kernel, nblocks=nblocks, n_total=n),
        grid=(2, nblocks),
        in_specs=[
            pl.BlockSpec((d, blk), lambda p, j: (0, j * p)),
            pl.BlockSpec((2, blk), lambda p, j: (0, j)),
            pl.BlockSpec((1, blk), lambda p, j: (0, j)),
            pl.BlockSpec((2 * C, 16), lambda p, j: (0, 0)),
        ],
        out_specs=[pl.BlockSpec((1, 1), lambda p, j: (0, 0))] * 6,
        out_shape=[scalar] * 6,
        scratch_shapes=[
            pltpu.VMEM((C, 3), jnp.float32),
            pltpu.VMEM((C, 16), jnp.float32),
            pltpu.VMEM((C, 1), jnp.float32),
            pltpu.VMEM((C, 1), jnp.float32),
            pltpu.VMEM((8, blk), jnp.float32),
            pltpu.VMEM((1, blk), jnp.float32),
        ],
    )(xt, st, lab, xpart)
    return tuple(o.reshape(()) for o in out)


# speedup vs baseline: 4.8575x; 4.8049x over previous
import functools
import math

import jax
import jax.numpy as jnp
from jax import lax
from jax.experimental import pallas as pl
from jax.experimental.pallas import tpu as pltpu

N_CLUSTERS = 32
DELTA_DIST = 1.5
W_VAR, W_DIST, W_REG, W_SMOOTH, W_SEED = 3.0, 1.0, 0.001, 5.0, 5.0

_LOG_LO = math.log(1e-12)
_PMAX = 1.0 - 1e-12


def _sum11(a):
    return jnp.sum(jnp.sum(a, axis=0, keepdims=True), axis=1, keepdims=True)


def _loss_kernel(x_ref, st_ref, lab_ref,
                 loss_ref, varl_ref, distl_ref, regl_ref, seedl_ref, smoothl_ref,
                 acc_all, cm_s, c2_s, i2v_s,
                 row_bce, row_seed,
                 *, nblocks, n_total, blk):
    phase = pl.program_id(0)
    j = pl.program_id(1)
    C = N_CLUSTERS
    f32 = jnp.float32

    B = blk
    sl = pl.ds(j * blk, blk)
    labels = lab_ref[:, sl]
    cid = lax.broadcasted_iota(jnp.int32, (C, B), 0)
    onehot = (cid == labels)

    @pl.when(jnp.logical_and(phase == 0, j == 0))
    def _init():
        acc_all[...] = jnp.zeros_like(acc_all)

    @pl.when(phase == 0)
    def _stats():
        x = x_ref[:, sl]
        e = jnp.exp(st_ref[1:2, sl])
        ones_b = jnp.ones((1, B), dtype=f32)
        aug = jnp.concatenate([x, ones_b, e, e * e], axis=0)
        oh = onehot.astype(f32)
        acc_all[...] += lax.dot_general(
            oh, aug, (((1,), (1,)), ((), ())), preferred_element_type=f32)

    @pl.when(jnp.logical_and(phase == 1, j == 0))
    def _finalize_stats():
        d = x_ref.shape[0]
        cnt = acc_all[:, d:d + 1]
        cm = acc_all[:, :d] / cnt
        cm_s[...] = cm
        c2_s[...] = jnp.sum(cm * cm, axis=1, keepdims=True)
        var_c = acc_all[:, d + 1:d + 2] / cnt
        i2v_s[...] = 0.5 / var_c

    @pl.when(phase == 1)
    def _dense():
        x = x_ref[:, sl]
        cm = cm_s[...]
        xc = lax.dot_general(
            cm, x, (((1,), (0,)), ((), ())), preferred_element_type=f32)
        xsq = x * x
        ones_d = jnp.ones((1, x.shape[0]), dtype=f32)
        x2 = lax.dot_general(
            ones_d, xsq, (((1,), (0,)), ((), ())), preferred_element_type=f32)
        d2 = jnp.maximum(x2 - 2.0 * xc + c2_s[...], 0.0)
        t = d2 * i2v_s[...]
        prob = jnp.exp(-t)
        p = jnp.minimum(prob, _PMAX)
        log1mp = jnp.log1p(-p)
        fold8 = (log1mp[0:8, :] + log1mp[8:16, :]
                 + log1mp[16:24, :] + log1mp[24:32, :])
        prob_own = jnp.sum(jnp.where(onehot, prob, 0.0), axis=0, keepdims=True)
        lp_own = jnp.maximum(jnp.log(prob_own), _LOG_LO)
        l1mp_own = jnp.log1p(-jnp.minimum(prob_own, _PMAX))
        keep = j != 0
        row_bce[...] = jnp.where(keep, row_bce[...], 0.0) + fold8
        row_bce[0:1, :] += lp_own - l1mp_own
        dsq = prob_own - st_ref[0:1, sl]
        row_seed[...] = jnp.where(keep, row_seed[...], 0.0) + dsq * dsq

    @pl.when(jnp.logical_and(phase == 1, j == nblocks - 1))
    def _combine():
        n = f32(n_total)
        d = x_ref.shape[0]
        cnt = acc_all[:, d:d + 1]
        cm = cm_s[...]
        c2col = c2_s[...]
        cmsq = cm * cm
        ones_d = jnp.ones((1, d), dtype=f32)
        c2row = lax.dot_general(
            ones_d, cmsq, (((1,), (1,)), ((), ())), preferred_element_type=f32)
        gram = lax.dot_general(
            cm, cm, (((1,), (1,)), ((), ())), preferred_element_type=f32)
        r = lax.broadcasted_iota(jnp.int32, (C, C), 0)
        cc = lax.broadcasted_iota(jnp.int32, (C, C), 1)
        eye = (r == cc).astype(f32)
        dmat = jnp.sqrt(jnp.maximum(c2col + c2row - 2.0 * gram, 0.0) + eye)
        hinge = jnp.maximum(2.0 * DELTA_DIST - dmat, 0.0)
        dist_loss = _sum11(hinge * hinge * (1.0 - eye)) / f32((C - 1) * C)
        reg_loss = _sum11(jnp.sqrt(c2col)) / f32(C)
        var_c = acc_all[:, d + 1:d + 2] / cnt
        smooth_c = acc_all[:, d + 2:d + 3] / cnt - var_c * var_c
        smoothness_loss = _sum11(smooth_c) / f32(C)
        var_loss = -_sum11(row_bce[...]) / (n * f32(C))
        seed_loss = _sum11(row_seed[...]) / n
        loss = (W_VAR * var_loss + W_DIST * dist_loss + W_REG * reg_loss
                + W_SMOOTH * smoothness_loss + W_SEED * seed_loss)
        loss_ref[...] = loss
        varl_ref[...] = W_VAR * var_loss
        distl_ref[...] = W_DIST * dist_loss
        regl_ref[...] = W_REG * reg_loss
        seedl_ref[...] = W_SEED * seed_loss
        smoothl_ref[...] = W_SMOOTH * smoothness_loss


@jax.jit
def kernel(features, seediness, group_labels):
    n, d = features.shape
    blk = 16384
    nblocks = n // blk
    xt = features.T
    st = seediness.T
    lab = group_labels.reshape(1, n).astype(jnp.int32)

    scalar = jax.ShapeDtypeStruct((1, 1), jnp.float32)
    out = pl.pallas_call(
        functools.partial(_loss_kernel, nblocks=nblocks, n_total=n, blk=blk),
        grid=(2, nblocks),
        in_specs=[
            pl.BlockSpec((d, n), lambda p, j: (0, 0)),
            pl.BlockSpec((2, n), lambda p, j: (0, 0)),
            pl.BlockSpec((1, n), lambda p, j: (0, 0)),
        ],
        out_specs=[pl.BlockSpec((1, 1), lambda p, j: (0, 0))] * 6,
        out_shape=[scalar] * 6,
        scratch_shapes=[
            pltpu.VMEM((N_CLUSTERS, 19), jnp.float32),
            pltpu.VMEM((N_CLUSTERS, 16), jnp.float32),
            pltpu.VMEM((N_CLUSTERS, 1), jnp.float32),
            pltpu.VMEM((N_CLUSTERS, 1), jnp.float32),
            pltpu.VMEM((8, blk), jnp.float32),
            pltpu.VMEM((1, blk), jnp.float32),
        ],
    )(xt, st, lab)
    return tuple(o.reshape(()) for o in out)


# speedup vs baseline: 4.9561x; 1.0203x over previous
import functools
import math

import jax
import jax.numpy as jnp
from jax import lax
from jax.experimental import pallas as pl
from jax.experimental.pallas import tpu as pltpu

N_CLUSTERS = 32
DELTA_DIST = 1.5
W_VAR, W_DIST, W_REG, W_SMOOTH, W_SEED = 3.0, 1.0, 0.001, 5.0, 5.0

_LOG_LO = math.log(1e-12)
_PMAX = 1.0 - 1e-12


def _sum11(a):
    return jnp.sum(jnp.sum(a, axis=0, keepdims=True), axis=1, keepdims=True)


def _loss_kernel(x_ref, st_ref, lab_ref,
                 loss_ref, varl_ref, distl_ref, regl_ref, seedl_ref, smoothl_ref,
                 acc_all, cm_s, c2_s, i2v_s,
                 row_bce, row_seed,
                 *, nblocks, n_total, blk):
    phase = pl.program_id(0)
    j = pl.program_id(1)
    C = N_CLUSTERS
    f32 = jnp.float32

    B = blk
    sl = pl.ds(j * blk, blk)
    labels = lab_ref[:, sl]
    cid = lax.broadcasted_iota(jnp.int32, (C, B), 0)
    onehot = (cid == labels)

    @pl.when(jnp.logical_and(phase == 0, j == 0))
    def _init():
        acc_all[...] = jnp.zeros_like(acc_all)

    @pl.when(phase == 0)
    def _stats():
        x = x_ref[:, sl]
        e = jnp.exp(st_ref[1:2, sl])
        ones_b = jnp.ones((1, B), dtype=f32)
        aug = jnp.concatenate([x, ones_b, e, e * e], axis=0)
        oh = onehot.astype(f32)
        acc_all[...] += lax.dot_general(
            oh, aug, (((1,), (1,)), ((), ())), preferred_element_type=f32)

    @pl.when(jnp.logical_and(phase == 1, j == 0))
    def _finalize_stats():
        d = x_ref.shape[0]
        cnt = acc_all[:, d:d + 1]
        cm = acc_all[:, :d] / cnt
        cm_s[...] = cm
        c2_s[...] = jnp.sum(cm * cm, axis=1, keepdims=True)
        var_c = acc_all[:, d + 1:d + 2] / cnt
        i2v_s[...] = 0.5 / var_c

    @pl.when(phase == 1)
    def _dense():
        x = x_ref[:, sl]
        cm = cm_s[...]
        xc = lax.dot_general(
            cm, x, (((1,), (0,)), ((), ())), preferred_element_type=f32)
        xsq = x * x
        ones_d = jnp.ones((1, x.shape[0]), dtype=f32)
        x2 = lax.dot_general(
            ones_d, xsq, (((1,), (0,)), ((), ())), preferred_element_type=f32)
        d2 = jnp.maximum(x2 - 2.0 * xc + c2_s[...], 0.0)
        t = d2 * i2v_s[...]
        prob = jnp.exp(-t)
        p = jnp.minimum(prob, _PMAX)
        log1mp = jnp.log1p(-p)
        fold8 = (log1mp[0:8, :] + log1mp[8:16, :]
                 + log1mp[16:24, :] + log1mp[24:32, :])
        prob_own = jnp.sum(jnp.where(onehot, prob, 0.0), axis=0, keepdims=True)
        lp_own = jnp.maximum(jnp.log(prob_own), _LOG_LO)
        l1mp_own = jnp.log1p(-jnp.minimum(prob_own, _PMAX))
        keep = j != 0
        row_bce[...] = jnp.where(keep, row_bce[...], 0.0) + fold8
        row_bce[0:1, :] += lp_own - l1mp_own
        dsq = prob_own - st_ref[0:1, sl]
        row_seed[...] = jnp.where(keep, row_seed[...], 0.0) + dsq * dsq

    @pl.when(jnp.logical_and(phase == 1, j == nblocks - 1))
    def _combine():
        n = f32(n_total)
        d = x_ref.shape[0]
        cnt = acc_all[:, d:d + 1]
        cm = cm_s[...]
        c2col = c2_s[...]
        cmsq = cm * cm
        ones_d = jnp.ones((1, d), dtype=f32)
        c2row = lax.dot_general(
            ones_d, cmsq, (((1,), (1,)), ((), ())), preferred_element_type=f32)
        gram = lax.dot_general(
            cm, cm, (((1,), (1,)), ((), ())), preferred_element_type=f32)
        r = lax.broadcasted_iota(jnp.int32, (C, C), 0)
        cc = lax.broadcasted_iota(jnp.int32, (C, C), 1)
        eye = (r == cc).astype(f32)
        dmat = jnp.sqrt(jnp.maximum(c2col + c2row - 2.0 * gram, 0.0) + eye)
        hinge = jnp.maximum(2.0 * DELTA_DIST - dmat, 0.0)
        dist_loss = _sum11(hinge * hinge * (1.0 - eye)) / f32((C - 1) * C)
        reg_loss = _sum11(jnp.sqrt(c2col)) / f32(C)
        var_c = acc_all[:, d + 1:d + 2] / cnt
        smooth_c = acc_all[:, d + 2:d + 3] / cnt - var_c * var_c
        smoothness_loss = _sum11(smooth_c) / f32(C)
        var_loss = -_sum11(row_bce[...]) / (n * f32(C))
        seed_loss = _sum11(row_seed[...]) / n
        loss = (W_VAR * var_loss + W_DIST * dist_loss + W_REG * reg_loss
                + W_SMOOTH * smoothness_loss + W_SEED * seed_loss)
        loss_ref[...] = loss
        varl_ref[...] = W_VAR * var_loss
        distl_ref[...] = W_DIST * dist_loss
        regl_ref[...] = W_REG * reg_loss
        seedl_ref[...] = W_SEED * seed_loss
        smoothl_ref[...] = W_SMOOTH * smoothness_loss


@jax.jit
def kernel(features, seediness, group_labels):
    n, d = features.shape
    blk = 32768
    nblocks = n // blk
    xt = features.T
    st = seediness.T
    lab = group_labels.reshape(1, n).astype(jnp.int32)

    scalar = jax.ShapeDtypeStruct((1, 1), jnp.float32)
    out = pl.pallas_call(
        functools.partial(_loss_kernel, nblocks=nblocks, n_total=n, blk=blk),
        grid=(2, nblocks),
        in_specs=[
            pl.BlockSpec((d, n), lambda p, j: (0, 0)),
            pl.BlockSpec((2, n), lambda p, j: (0, 0)),
            pl.BlockSpec((1, n), lambda p, j: (0, 0)),
        ],
        out_specs=[pl.BlockSpec((1, 1), lambda p, j: (0, 0))] * 6,
        out_shape=[scalar] * 6,
        scratch_shapes=[
            pltpu.VMEM((N_CLUSTERS, 19), jnp.float32),
            pltpu.VMEM((N_CLUSTERS, 16), jnp.float32),
            pltpu.VMEM((N_CLUSTERS, 1), jnp.float32),
            pltpu.VMEM((N_CLUSTERS, 1), jnp.float32),
            pltpu.VMEM((8, blk), jnp.float32),
            pltpu.VMEM((1, blk), jnp.float32),
        ],
    )(xt, st, lab)
    return tuple(o.reshape(()) for o in out)


# speedup vs baseline: 5.0880x; 1.0266x over previous
import functools
import math

import jax
import jax.numpy as jnp
from jax import lax
from jax.experimental import pallas as pl
from jax.experimental.pallas import tpu as pltpu

N_CLUSTERS = 32
DELTA_DIST = 1.5
W_VAR, W_DIST, W_REG, W_SMOOTH, W_SEED = 3.0, 1.0, 0.001, 5.0, 5.0

_LOG_LO = math.log(1e-12)
_PMAX = 1.0 - 1e-12


def _sum11(a):
    return jnp.sum(jnp.sum(a, axis=0, keepdims=True), axis=1, keepdims=True)


def _loss_kernel(x_ref, st_ref, lab_ref,
                 loss_ref, varl_ref, distl_ref, regl_ref, seedl_ref, smoothl_ref,
                 acc_all, cm_s, c2_s, coef_s,
                 row_bce, row_seed,
                 *, nblocks, n_total, blk):
    phase = pl.program_id(0)
    j = pl.program_id(1)
    C = N_CLUSTERS
    f32 = jnp.float32

    B = blk
    sl = pl.ds(j * blk, blk)
    labels = lab_ref[:, sl]
    cid = lax.broadcasted_iota(jnp.int32, (C, B), 0)
    onehot = (cid == labels)

    @pl.when(jnp.logical_and(phase == 0, j == 0))
    def _init():
        acc_all[...] = jnp.zeros_like(acc_all)

    @pl.when(phase == 0)
    def _stats():
        x = x_ref[:, sl]
        e = jnp.exp(st_ref[1:2, sl])
        ones_b = jnp.ones((1, B), dtype=f32)
        aug = jnp.concatenate([x, ones_b, e, e * e], axis=0)
        oh = onehot.astype(f32)
        acc_all[...] += lax.dot_general(
            oh, aug, (((1,), (1,)), ((), ())), preferred_element_type=f32)

    @pl.when(jnp.logical_and(phase == 1, j == 0))
    def _finalize_stats():
        d = x_ref.shape[0]
        cnt = acc_all[:, d:d + 1]
        cm = acc_all[:, :d] / cnt
        cm_s[...] = cm
        c2 = jnp.sum(cm * cm, axis=1, keepdims=True)
        c2_s[...] = c2
        var_c = acc_all[:, d + 1:d + 2] / cnt
        i2v = 0.5 / var_c
        coef_s[...] = jnp.concatenate([-i2v, 2.0 * i2v, -c2 * i2v], axis=1)

    @pl.when(phase == 1)
    def _dense():
        x = x_ref[:, sl]
        cm = cm_s[...]
        xc = lax.dot_general(
            cm, x, (((1,), (0,)), ((), ())), preferred_element_type=f32)
        xsq = x * x
        ones_d = jnp.ones((1, x.shape[0]), dtype=f32)
        x2 = lax.dot_general(
            ones_d, xsq, (((1,), (0,)), ((), ())), preferred_element_type=f32)
        nt = (x2 * coef_s[:, 0:1] + xc * coef_s[:, 1:2]) + coef_s[:, 2:3]
        prob = jnp.exp(nt)
        p = jnp.minimum(prob, _PMAX)
        log1mp = jnp.log1p(-p)
        fold8 = (log1mp[0:8, :] + log1mp[8:16, :]
                 + log1mp[16:24, :] + log1mp[24:32, :])
        prob_own = jnp.sum(jnp.where(onehot, prob, 0.0), axis=0, keepdims=True)
        lp_own = jnp.maximum(jnp.log(prob_own), _LOG_LO)
        l1mp_own = jnp.log1p(-jnp.minimum(prob_own, _PMAX))
        keep = j != 0
        row_bce[...] = jnp.where(keep, row_bce[...], 0.0) + fold8
        row_bce[0:1, :] += lp_own - l1mp_own
        dsq = prob_own - st_ref[0:1, sl]
        row_seed[...] = jnp.where(keep, row_seed[...], 0.0) + dsq * dsq

    @pl.when(jnp.logical_and(phase == 1, j == nblocks - 1))
    def _combine():
        n = f32(n_total)
        d = x_ref.shape[0]
        cnt = acc_all[:, d:d + 1]
        cm = cm_s[...]
        c2col = c2_s[...]
        cmsq = cm * cm
        ones_d = jnp.ones((1, d), dtype=f32)
        c2row = lax.dot_general(
            ones_d, cmsq, (((1,), (1,)), ((), ())), preferred_element_type=f32)
        gram = lax.dot_general(
            cm, cm, (((1,), (1,)), ((), ())), preferred_element_type=f32)
        r = lax.broadcasted_iota(jnp.int32, (C, C), 0)
        cc = lax.broadcasted_iota(jnp.int32, (C, C), 1)
        eye = (r == cc).astype(f32)
        dmat = jnp.sqrt(jnp.maximum(c2col + c2row - 2.0 * gram, 0.0) + eye)
        hinge = jnp.maximum(2.0 * DELTA_DIST - dmat, 0.0)
        dist_loss = _sum11(hinge * hinge * (1.0 - eye)) / f32((C - 1) * C)
        reg_loss = _sum11(jnp.sqrt(c2col)) / f32(C)
        var_c = acc_all[:, d + 1:d + 2] / cnt
        smooth_c = acc_all[:, d + 2:d + 3] / cnt - var_c * var_c
        smoothness_loss = _sum11(smooth_c) / f32(C)
        var_loss = -_sum11(row_bce[...]) / (n * f32(C))
        seed_loss = _sum11(row_seed[...]) / n
        loss = (W_VAR * var_loss + W_DIST * dist_loss + W_REG * reg_loss
                + W_SMOOTH * smoothness_loss + W_SEED * seed_loss)
        loss_ref[...] = loss
        varl_ref[...] = W_VAR * var_loss
        distl_ref[...] = W_DIST * dist_loss
        regl_ref[...] = W_REG * reg_loss
        seedl_ref[...] = W_SEED * seed_loss
        smoothl_ref[...] = W_SMOOTH * smoothness_loss


@jax.jit
def kernel(features, seediness, group_labels):
    n, d = features.shape
    blk = 32768
    nblocks = n // blk
    xt = features.T
    st = seediness.T
    lab = group_labels.reshape(1, n).astype(jnp.int32)

    scalar = jax.ShapeDtypeStruct((1, 1), jnp.float32)
    out = pl.pallas_call(
        functools.partial(_loss_kernel, nblocks=nblocks, n_total=n, blk=blk),
        grid=(2, nblocks),
        in_specs=[
            pl.BlockSpec((d, n), lambda p, j: (0, 0)),
            pl.BlockSpec((2, n), lambda p, j: (0, 0)),
            pl.BlockSpec((1, n), lambda p, j: (0, 0)),
        ],
        out_specs=[pl.BlockSpec((1, 1), lambda p, j: (0, 0))] * 6,
        out_shape=[scalar] * 6,
        scratch_shapes=[
            pltpu.VMEM((N_CLUSTERS, 19), jnp.float32),
            pltpu.VMEM((N_CLUSTERS, 16), jnp.float32),
            pltpu.VMEM((N_CLUSTERS, 1), jnp.float32),
            pltpu.VMEM((N_CLUSTERS, 3), jnp.float32),
            pltpu.VMEM((8, blk), jnp.float32),
            pltpu.VMEM((1, blk), jnp.float32),
        ],
    )(xt, st, lab)
    return tuple(o.reshape(()) for o in out)
